# baseline (speedup 1.0000x reference)
import functools

import jax
import jax.numpy as jnp
from jax import lax
from jax.experimental import pallas as pl
from jax.experimental.pallas import tpu as pltpu
from jax.experimental.pallas import tpu_sc as plsc

TINY = 1e-6
M = 128
N = 64
BATCH = 16384
NC, NS, L = 2, 16, 16
NW = NC * NS
RAYS_PER_W = BATCH // NW
G = 64
NBATCH = RAYS_PER_W // G
NGRP = G // L
DELTA = 4.0 / 127.0


def _body(od_hbm, w_hbm, pts_hbm, z_hbm, s_hbm,
          w_v, od_v, cdf_v, hist_v, cb_v, pb_v, pts_v, z_v,
          sem_in0, sem_in1, sem_out0, sem_out1):
    wid = lax.axis_index("s") * NC + lax.axis_index("c")
    iota = lax.iota(jnp.int32, L)
    zero_f = jnp.zeros((L,), jnp.float32)
    ones_f = jnp.full((L,), 1.0, jnp.float32)
    sem_in = (sem_in0, sem_in1)
    sem_out = (sem_out0, sem_out1)

    def in_copy(g, b):
        return pltpu.make_async_copy(
            w_hbm.at[pl.ds(wid * RAYS_PER_W + g * G, G)], w_v.at[b],
            sem_in[b])

    def out_copies(g, b):
        base = wid * RAYS_PER_W + g * G
        return (
            pltpu.make_async_copy(
                pts_v.at[b], pts_hbm.at[:, pl.ds(base, G)], sem_out[b]),
            pltpu.make_async_copy(
                z_v.at[b], z_hbm.at[:, pl.ds(base, G)], sem_out[b]),
            pltpu.make_async_copy(
                z_v.at[b], s_hbm.at[:, pl.ds(base, G)], sem_out[b]),
        )

    for p in range(6):
        pltpu.sync_copy(od_hbm.at[pl.ds(p * (BATCH // 128) + wid * 4, 4)],
                        od_v.at[p])

    in_copy(0, 0).start()

    def compute_batch(g, b):
        wb = w_v.at[b]

        def zero_body(r, zc):
            for grp in range(NGRP):
                hist_v[grp, r, :] = zero_f
                cb_v[grp, r, :] = zero_f
                pb_v[grp, r, :] = zero_f
            return zc
        lax.fori_loop(0, N + 1, zero_body, 0, unroll=4)

        def cdf_body(m_, cs):
            mvec = jnp.full((L,), m_, jnp.int32)
            out = []
            for grp in range(NGRP):
                wv = plsc.load_gather(wb, [iota + grp * L, mvec])
                c = cs[grp] + (wv + TINY)
                cdf_v[grp, m_, :] = c
                out.append(c)
            return tuple(out)
        totals = lax.fori_loop(0, M, cdf_body, (zero_f,) * NGRP, unroll=8)
        recips = [1.0 / t for t in totals]
        s63rs = [63.0 * r for r in recips]

        ods = []
        for grp in range(NGRP):
            rl = g * G + grp * L
            row = lax.shift_right_logical(rl, 7)
            col = lax.bitwise_and(rl, 127)
            ods.append([od_v[p, row, pl.ds(col, L)] for p in range(6)])

        def scat_body(m_, cs):
            out = []
            for grp in range(NGRP):
                c_cur = cs[grp]
                c_next = cdf_v[grp, m_ + 1, :]
                x = c_cur * s63rs[grp]
                ti = x.astype(jnp.int32)
                ti = ti + jnp.where(ti.astype(jnp.float32) < x, 1, 0)
                ti = jnp.minimum(ti, N)
                plsc.addupdate_scatter(hist_v.at[grp], [ti, iota], ones_f)
                plsc.store_scatter(cb_v.at[grp], [ti, iota], c_cur)
                plsc.store_scatter(pb_v.at[grp], [ti, iota], c_next - c_cur)
                out.append(c_next)
            return tuple(out)
        c0s = tuple(cdf_v[grp, 0, :] for grp in range(NGRP))
        lax.fori_loop(0, M - 1, scat_body, c0s, unroll=8)

        def sample_body(n_, st):
            belows, cbmaxs, pds = st
            u = jnp.full((L,), n_, jnp.int32).astype(jnp.float32) * (1.0 / 63.0)
            nb, ncb, npd = [], [], []
            for grp in range(NGRP):
                h = hist_v[grp, n_, :]
                below_f = belows[grp] + h
                cbmax = jnp.maximum(cbmaxs[grp], cb_v[grp, n_, :])
                pd = jnp.where(h > 0.0, pb_v[grp, n_, :], pds[grp])
                recip = recips[grp]
                cB = cbmax * recip
                denom = pd * recip
                denom = jnp.where(denom < TINY, 1.0, denom)
                t = (u - cB) / denom
                blo = jnp.clip(below_f - 0.5, 0.0, 127.0)
                bhi = jnp.minimum(below_f + 0.5, 127.0)
                samples = 2.0 + blo * DELTA + t * ((bhi - blo) * DELTA + TINY)
                z_v[b, n_, pl.ds(grp * L, L)] = samples
                ox, oy, oz, dx, dy, dz = ods[grp]
                for cmp_i, (o_s, d_s) in enumerate(
                        ((ox, dx), (oy, dy), (oz, dz))):
                    pts_v[b, cmp_i * N + n_, pl.ds(grp * L, L)] = (
                        o_s + d_s * samples)
                nb.append(below_f)
                ncb.append(cbmax)
                npd.append(pd)
            return (tuple(nb), tuple(ncb), tuple(npd))
        lax.fori_loop(0, N, sample_body,
                      ((zero_f,) * NGRP, (zero_f,) * NGRP, c0s), unroll=8)

    def pair_body(i, carry):
        for b in range(2):
            g = i * 2 + b
            in_copy(g, b).wait()

            @pl.when(g + 1 < NBATCH)
            def _():
                in_copy(g + 1, 1 - b).start()

            @pl.when(g >= 2)
            def _():
                for cp in out_copies(g - 2, b):
                    cp.wait()

            compute_batch(g, b)
            for cp in out_copies(g, b):
                cp.start()
        return carry

    lax.fori_loop(0, NBATCH // 2, pair_body, 0, unroll=False)
    for cp in out_copies(NBATCH - 2, 0):
        cp.wait()
    for cp in out_copies(NBATCH - 1, 1):
        cp.wait()


@jax.jit
def kernel(rays_o, rays_d, weights):
    mesh = plsc.VectorSubcoreMesh(core_axis_name="c", subcore_axis_name="s")
    f = pl.kernel(
        _body,
        out_type=(
            jax.ShapeDtypeStruct((3 * N, BATCH), jnp.float32),
            jax.ShapeDtypeStruct((N, BATCH), jnp.float32),
            jax.ShapeDtypeStruct((N, BATCH), jnp.float32),
        ),
        mesh=mesh,
        compiler_params=pltpu.CompilerParams(
            needs_layout_passes=False, use_tc_tiling_on_sc=False),
        scratch_types=[
            pltpu.VMEM((2, G, M), jnp.float32),
            pltpu.VMEM((6, 4, 128), jnp.float32),
            pltpu.VMEM((NGRP, M, L), jnp.float32),
            pltpu.VMEM((NGRP, N + 1, L), jnp.float32),
            pltpu.VMEM((NGRP, N + 1, L), jnp.float32),
            pltpu.VMEM((NGRP, N + 1, L), jnp.float32),
            pltpu.VMEM((2, 3 * N, G), jnp.float32),
            pltpu.VMEM((2, N, G), jnp.float32),
            pltpu.SemaphoreType.DMA,
            pltpu.SemaphoreType.DMA,
            pltpu.SemaphoreType.DMA,
            pltpu.SemaphoreType.DMA,
        ],
    )
    od = jnp.reshape(
        jnp.transpose(jnp.concatenate([rays_o, rays_d], axis=1)),
        (6 * BATCH // 128, 128))
    pts_t, z_t, s_t = f(od, weights)
    pts = jnp.transpose(jnp.reshape(pts_t, (3, N, BATCH)), (2, 1, 0))
    return (pts, jnp.transpose(z_t), jnp.transpose(s_t))
